# baseline (speedup 1.0000x reference)
import functools

import jax
import jax.numpy as jnp
from jax import lax
from jax.experimental import pallas as pl
from jax.experimental.pallas import tpu as pltpu
from jax.experimental.pallas import tpu_sc as plsc

NC = 2
NS = 16
NW = NC * NS
CH = 128
DC = 32
DP = 128


def _lin0_body(x_ref, w_ref, b_ref, o_ref):
    o_ref[...] = jnp.maximum(
        jnp.dot(x_ref[...], w_ref[...], preferred_element_type=jnp.float32)
        + b_ref[...], 0.0)


def _ew_body(ea_ref, we1_ref, be1_ref, we2_ref, be2_ref, o_ref):
    h1 = jnp.maximum(
        jnp.dot(ea_ref[...], we1_ref[...], preferred_element_type=jnp.float32)
        + be1_ref[...], 0.0)
    o_ref[...] = (jnp.dot(h1, we2_ref[...], preferred_element_type=jnp.float32)
                  + be2_ref[...])


def _msg_body(s_ref, ew_ref, sel_ref, o_ref):
    s = s_ref[:, 0:DC]
    srep = jnp.concatenate([s] * DC, axis=1)
    t = ew_ref[...] * srep
    msg = jnp.dot(t, sel_ref[...], preferred_element_type=jnp.float32)
    o_ref[...] = jnp.concatenate(
        [msg, jnp.zeros((msg.shape[0], DP - DC), jnp.float32)], axis=1)


def _gru_body(h_ref, agg_ref, deg_ref, wroot_ref, bconv_ref,
              wih_ref, whh_ref, bih_ref, bhh_ref, o_ref):
    h = h_ref[:, 0:DC]
    agg = agg_ref[0, :, 0:DC] + agg_ref[1, :, 0:DC]
    deg = jnp.maximum(deg_ref[0, :, 0:DC] + deg_ref[1, :, 0:DC], 1.0)
    m = jnp.maximum(
        jnp.dot(h, wroot_ref[...], preferred_element_type=jnp.float32)
        + agg / deg + bconv_ref[...], 0.0)
    gi = jnp.dot(m, wih_ref[...], preferred_element_type=jnp.float32) + bih_ref[...]
    gh = jnp.dot(h, whh_ref[...], preferred_element_type=jnp.float32) + bhh_ref[...]
    r = jax.nn.sigmoid(gi[:, 0:DC] + gh[:, 0:DC])
    z = jax.nn.sigmoid(gi[:, DC:2 * DC] + gh[:, DC:2 * DC])
    ng = jnp.tanh(gi[:, 2 * DC:3 * DC] + r * gh[:, 2 * DC:3 * DC])
    hn = (1.0 - z) * ng + z * h
    o_ref[...] = jnp.concatenate(
        [hn, jnp.zeros((hn.shape[0], DP - DC), jnp.float32)], axis=1)


def _s2s_body(h_ref, bcol_ref, ga_ref, wih_ref, whh_ref, bih_ref, bhh_ref,
              w1_ref, b1_ref, w2_ref, b2_ref, o_ref):
    h = h_ref[:, 0:DC]
    n = h.shape[0]
    ng = ga_ref.shape[0]
    m2 = bcol_ref[...] == lax.broadcasted_iota(jnp.int32, (n, ng), 1)
    m2f = m2.astype(jnp.float32)
    qh = jnp.zeros((ng, DC), jnp.float32)
    qc = jnp.zeros((ng, DC), jnp.float32)
    q_star = jnp.zeros((ng, 2 * DC), jnp.float32)
    for _ in range(3):
        gates = (jnp.dot(q_star, wih_ref[...], preferred_element_type=jnp.float32)
                 + bih_ref[...]
                 + jnp.dot(qh, whh_ref[...], preferred_element_type=jnp.float32)
                 + bhh_ref[...])
        ig = jax.nn.sigmoid(gates[:, 0:DC])
        fg = jax.nn.sigmoid(gates[:, DC:2 * DC])
        gg = jnp.tanh(gates[:, 2 * DC:3 * DC])
        og = jax.nn.sigmoid(gates[:, 3 * DC:4 * DC])
        qc = fg * qc + ig * gg
        qh = og * jnp.tanh(qc)
        e2 = lax.dot_general(h, qh, (((1,), (1,)), ((), ())),
                             preferred_element_type=jnp.float32)
        e = jnp.sum(jnp.where(m2, e2, 0.0), axis=1, keepdims=True)
        emax = jnp.max(jnp.where(m2, e, -1e30), axis=0, keepdims=True)
        ee = jnp.exp(e - jnp.sum(m2f * emax, axis=1, keepdims=True))
        denom = jnp.sum(m2f * ee, axis=0, keepdims=True)
        a = ee / (jnp.sum(m2f * denom, axis=1, keepdims=True) + 1e-16)
        rvec = lax.dot_general(m2f * a, h, (((0,), (0,)), ((), ())),
                               preferred_element_type=jnp.float32)
        q_star = jnp.concatenate([qh, rvec], axis=1)
    og2 = jnp.concatenate([q_star, ga_ref[...]], axis=1)
    og2 = jnp.maximum(
        jnp.dot(og2, w1_ref[...], preferred_element_type=jnp.float32)
        + b1_ref[...], 0.0)
    o_ref[...] = (jnp.dot(og2, w2_ref[...], preferred_element_type=jnp.float32)
                  + b2_ref[...])


def _sc_mesh():
    return plsc.VectorSubcoreMesh(core_axis_name="c", subcore_axis_name="s")


@functools.lru_cache(maxsize=None)
def _make_gather(ep):
    nchunk = ep // (CH * NW)

    @functools.partial(
        pl.kernel,
        out_type=jax.ShapeDtypeStruct((ep, DP), jnp.float32),
        mesh=_sc_mesh(),
        scratch_types=[pltpu.VMEM((CH,), jnp.int32),
                       pltpu.VMEM((CH, DP), jnp.float32),
                       pltpu.SemaphoreType.DMA],
    )
    def gather_k(table_hbm, src_hbm, out_hbm, idx_v, rows_v, sem):
        wid = lax.axis_index("s") * NC + lax.axis_index("c")

        def body(r, carry):
            off = (r * NW + wid) * CH
            pltpu.sync_copy(src_hbm.at[pl.ds(off, CH)], idx_v)
            pltpu.async_copy(table_hbm.at[idx_v], rows_v, sem).wait()
            pltpu.sync_copy(rows_v, out_hbm.at[pl.ds(off, CH)])
            return carry

        lax.fori_loop(0, nchunk, body, 0)

    return gather_k


@functools.lru_cache(maxsize=None)
def _make_scatter(ep, nr):
    nchunk = ep // (CH * NW)
    rps = nr // NS

    @functools.partial(
        pl.kernel,
        out_type=jax.ShapeDtypeStruct((NC * nr, DP), jnp.float32),
        mesh=_sc_mesh(),
        scratch_types=[pltpu.VMEM((CH,), jnp.int32),
                       pltpu.VMEM((CH, DP), jnp.float32),
                       pltpu.VMEM_SHARED((nr, DP), jnp.float32),
                       pltpu.SemaphoreType.DMA],
    )
    def scatter_k(msg_hbm, dst_hbm, zeros_hbm, out_hbm, idx_v, rows_v, shared, sem):
        c = lax.axis_index("c")
        s = lax.axis_index("s")
        wid = s * NC + c
        pltpu.sync_copy(zeros_hbm.at[pl.ds(s * rps, rps)],
                        shared.at[pl.ds(s * rps, rps)])
        plsc.subcore_barrier()

        def body(r, carry):
            off = (r * NW + wid) * CH
            pltpu.sync_copy(dst_hbm.at[pl.ds(off, CH)], idx_v)
            pltpu.sync_copy(msg_hbm.at[pl.ds(off, CH)], rows_v)
            pltpu.sync_copy(rows_v, shared.at[idx_v], add=True)
            return carry

        lax.fori_loop(0, nchunk, body, 0)
        plsc.subcore_barrier()
        pltpu.sync_copy(shared.at[pl.ds(s * rps, rps)],
                        out_hbm.at[pl.ds(c * nr + s * rps, rps)])

    return scatter_k


@functools.lru_cache(maxsize=None)
def _make_degree(ep, nr):
    nchunk = ep // (CH * NW)
    rps = nr // NS

    @functools.partial(
        pl.kernel,
        out_type=jax.ShapeDtypeStruct((NC * nr, DP), jnp.float32),
        mesh=_sc_mesh(),
        scratch_types=[pltpu.VMEM((CH,), jnp.int32),
                       pltpu.VMEM((CH, DP), jnp.float32),
                       pltpu.VMEM_SHARED((nr, DP), jnp.float32),
                       pltpu.SemaphoreType.DMA],
    )
    def degree_k(dst_hbm, zeros_hbm, ones_hbm, out_hbm, idx_v, rows_v, shared, sem):
        c = lax.axis_index("c")
        s = lax.axis_index("s")
        wid = s * NC + c
        pltpu.sync_copy(zeros_hbm.at[pl.ds(s * rps, rps)],
                        shared.at[pl.ds(s * rps, rps)])
        pltpu.sync_copy(ones_hbm, rows_v)
        plsc.subcore_barrier()

        def body(r, carry):
            off = (r * NW + wid) * CH
            pltpu.sync_copy(dst_hbm.at[pl.ds(off, CH)], idx_v)
            pltpu.sync_copy(rows_v, shared.at[idx_v], add=True)
            return carry

        lax.fori_loop(0, nchunk, body, 0)
        plsc.subcore_barrier()
        pltpu.sync_copy(shared.at[pl.ds(s * rps, rps)],
                        out_hbm.at[pl.ds(c * nr + s * rps, rps)])

    return degree_k


def kernel(x, edge_index, edge_attr, batch, graph_attr, W0, b0, We1, be1,
           We2, be2, Wroot, bconv, gru_Wih, gru_Whh, gru_bih, gru_bhh,
           lstm_Wih, lstm_Whh, lstm_bih, lstm_bhh, W1, b1, W2, b2):
    n, df = x.shape
    e = edge_index.shape[1]
    de = edge_attr.shape[1]
    ng, dg = graph_attr.shape

    ep = -(-e // (CH * NW)) * (CH * NW)
    nr = -(-(n + 1) // 128) * 128

    src_p = jnp.concatenate([edge_index[0], jnp.zeros((ep - e,), jnp.int32)])
    dst_p = jnp.concatenate([edge_index[1], jnp.full((ep - e,), n, jnp.int32)])
    ea_p = jnp.concatenate([edge_attr, jnp.zeros((ep - e, de), jnp.float32)])
    zeros_nr = jnp.zeros((nr, DP), jnp.float32)
    ones_ch = jnp.ones((CH, DP), jnp.float32)

    w0_p = jnp.concatenate([W0, jnp.zeros((df, DP - DC), jnp.float32)], axis=1)
    b0_p = jnp.concatenate([b0, jnp.zeros((DP - DC,), jnp.float32)]).reshape(1, DP)

    bl = 1000
    nb = n // bl
    bke = 1024
    neb = ep // bke

    h = pl.pallas_call(
        _lin0_body,
        grid=(nb,),
        in_specs=[pl.BlockSpec((bl, df), lambda i: (i, 0)),
                  pl.BlockSpec((df, DP), lambda i: (0, 0)),
                  pl.BlockSpec((1, DP), lambda i: (0, 0))],
        out_specs=pl.BlockSpec((bl, DP), lambda i: (i, 0)),
        out_shape=jax.ShapeDtypeStruct((n, DP), jnp.float32),
    )(x, w0_p, b0_p)

    perm = (jnp.arange(DC * DC) % DC) * DC + jnp.arange(DC * DC) // DC
    we2g = We2[:, perm]
    be2g = be2[perm]
    sel = jnp.kron(jnp.eye(DC, dtype=jnp.float32), jnp.ones((DC, 1), jnp.float32))
    ew = pl.pallas_call(
        _ew_body,
        grid=(neb,),
        in_specs=[pl.BlockSpec((bke, de), lambda i: (i, 0)),
                  pl.BlockSpec((de, We1.shape[1]), lambda i: (0, 0)),
                  pl.BlockSpec((1, We1.shape[1]), lambda i: (0, 0)),
                  pl.BlockSpec((We2.shape[0], DC * DC), lambda i: (0, 0)),
                  pl.BlockSpec((1, DC * DC), lambda i: (0, 0))],
        out_specs=pl.BlockSpec((bke, DC * DC), lambda i: (i, 0)),
        out_shape=jax.ShapeDtypeStruct((ep, DC * DC), jnp.float32),
    )(ea_p, We1, be1.reshape(1, -1), we2g, be2g.reshape(1, -1))

    deg = _make_degree(ep, nr)(dst_p, zeros_nr, ones_ch).reshape(NC, nr, DP)

    gather_k = _make_gather(ep)
    scatter_k = _make_scatter(ep, nr)

    wih_t = gru_Wih.T
    whh_t = gru_Whh.T
    gbih = gru_bih.reshape(1, -1)
    gbhh = gru_bhh.reshape(1, -1)

    for _ in range(3):
        s_rows = gather_k(h, src_p)
        msg = pl.pallas_call(
            _msg_body,
            grid=(neb,),
            in_specs=[pl.BlockSpec((bke, DP), lambda i: (i, 0)),
                      pl.BlockSpec((bke, DC * DC), lambda i: (i, 0)),
                      pl.BlockSpec((DC * DC, DC), lambda i: (0, 0))],
            out_specs=pl.BlockSpec((bke, DP), lambda i: (i, 0)),
            out_shape=jax.ShapeDtypeStruct((ep, DP), jnp.float32),
        )(s_rows, ew, sel)
        agg = scatter_k(msg, dst_p, zeros_nr).reshape(NC, nr, DP)
        h = pl.pallas_call(
            _gru_body,
            grid=(nb,),
            in_specs=[pl.BlockSpec((bl, DP), lambda i: (i, 0)),
                      pl.BlockSpec((NC, bl, DP), lambda i: (0, i, 0)),
                      pl.BlockSpec((NC, bl, DP), lambda i: (0, i, 0)),
                      pl.BlockSpec((DC, DC), lambda i: (0, 0)),
                      pl.BlockSpec((1, DC), lambda i: (0, 0)),
                      pl.BlockSpec((DC, 3 * DC), lambda i: (0, 0)),
                      pl.BlockSpec((DC, 3 * DC), lambda i: (0, 0)),
                      pl.BlockSpec((1, 3 * DC), lambda i: (0, 0)),
                      pl.BlockSpec((1, 3 * DC), lambda i: (0, 0))],
            out_specs=pl.BlockSpec((bl, DP), lambda i: (i, 0)),
            out_shape=jax.ShapeDtypeStruct((n, DP), jnp.float32),
        )(h, agg, deg, Wroot, bconv.reshape(1, DC), wih_t, whh_t, gbih, gbhh)

    out = pl.pallas_call(
        _s2s_body,
        out_shape=jax.ShapeDtypeStruct((ng, 1), jnp.float32),
    )(h, batch.reshape(n, 1), graph_attr, lstm_Wih.T, lstm_Whh.T,
      lstm_bih.reshape(1, -1), lstm_bhh.reshape(1, -1),
      W1, b1.reshape(1, -1), W2, b2.reshape(1, 1))
    return out


# speedup vs baseline: 1.1142x; 1.1142x over previous
import functools

import jax
import jax.numpy as jnp
from jax import lax
from jax.experimental import pallas as pl
from jax.experimental.pallas import tpu as pltpu
from jax.experimental.pallas import tpu_sc as plsc

NC = 2
NS = 16
NW = NC * NS
CH = 128
DC = 32
DP = 128


def _lin0_body(x_ref, w_ref, b_ref, o_ref):
    o_ref[...] = jnp.maximum(
        jnp.dot(x_ref[...], w_ref[...], preferred_element_type=jnp.float32)
        + b_ref[...], 0.0)


def _ew_body(ea_ref, we1_ref, be1_ref, we2_ref, be2_ref, o_ref):
    h1 = jnp.maximum(
        jnp.dot(ea_ref[...], we1_ref[...], preferred_element_type=jnp.float32)
        + be1_ref[...], 0.0)
    o_ref[...] = (jnp.dot(h1, we2_ref[...], preferred_element_type=jnp.float32)
                  + be2_ref[...]).astype(jnp.bfloat16)


def _msg_body(s_ref, ew_ref, sel_ref, o_ref):
    s = s_ref[:, 0:DC]
    srep = jnp.concatenate([s] * DC, axis=1)
    t = ew_ref[...].astype(jnp.float32) * srep
    msg = jnp.dot(t, sel_ref[...], preferred_element_type=jnp.float32)
    o_ref[...] = jnp.concatenate(
        [msg, jnp.zeros((msg.shape[0], DP - DC), jnp.float32)], axis=1)


def _gru_body(h_ref, agg_ref, deg_ref, wroot_ref, bconv_ref,
              wih_ref, whh_ref, bih_ref, bhh_ref, o_ref):
    h = h_ref[:, 0:DC]
    agg = agg_ref[0, :, 0:DC] + agg_ref[1, :, 0:DC]
    deg = jnp.maximum(deg_ref[0, :, 0:DC] + deg_ref[1, :, 0:DC], 1.0)
    m = jnp.maximum(
        jnp.dot(h, wroot_ref[...], preferred_element_type=jnp.float32)
        + agg / deg + bconv_ref[...], 0.0)
    gi = jnp.dot(m, wih_ref[...], preferred_element_type=jnp.float32) + bih_ref[...]
    gh = jnp.dot(h, whh_ref[...], preferred_element_type=jnp.float32) + bhh_ref[...]
    r = jax.nn.sigmoid(gi[:, 0:DC] + gh[:, 0:DC])
    z = jax.nn.sigmoid(gi[:, DC:2 * DC] + gh[:, DC:2 * DC])
    ng = jnp.tanh(gi[:, 2 * DC:3 * DC] + r * gh[:, 2 * DC:3 * DC])
    hn = (1.0 - z) * ng + z * h
    o_ref[...] = jnp.concatenate(
        [hn, jnp.zeros((hn.shape[0], DP - DC), jnp.float32)], axis=1)


def _s2s_body(h_ref, bcol_ref, ga_ref, wih_ref, whh_ref, bih_ref, bhh_ref,
              w1_ref, b1_ref, w2_ref, b2_ref, o_ref):
    h = h_ref[:, 0:DC]
    n = h.shape[0]
    ng = ga_ref.shape[0]
    m2 = bcol_ref[...] == lax.broadcasted_iota(jnp.int32, (n, ng), 1)
    m2f = m2.astype(jnp.float32)
    qh = jnp.zeros((ng, DC), jnp.float32)
    qc = jnp.zeros((ng, DC), jnp.float32)
    q_star = jnp.zeros((ng, 2 * DC), jnp.float32)
    for _ in range(3):
        gates = (jnp.dot(q_star, wih_ref[...], preferred_element_type=jnp.float32)
                 + bih_ref[...]
                 + jnp.dot(qh, whh_ref[...], preferred_element_type=jnp.float32)
                 + bhh_ref[...])
        ig = jax.nn.sigmoid(gates[:, 0:DC])
        fg = jax.nn.sigmoid(gates[:, DC:2 * DC])
        gg = jnp.tanh(gates[:, 2 * DC:3 * DC])
        og = jax.nn.sigmoid(gates[:, 3 * DC:4 * DC])
        qc = fg * qc + ig * gg
        qh = og * jnp.tanh(qc)
        e2 = lax.dot_general(h, qh, (((1,), (1,)), ((), ())),
                             preferred_element_type=jnp.float32)
        e = jnp.sum(jnp.where(m2, e2, 0.0), axis=1, keepdims=True)
        emax = jnp.max(jnp.where(m2, e, -1e30), axis=0, keepdims=True)
        ee = jnp.exp(e - jnp.sum(m2f * emax, axis=1, keepdims=True))
        denom = jnp.sum(m2f * ee, axis=0, keepdims=True)
        a = ee / (jnp.sum(m2f * denom, axis=1, keepdims=True) + 1e-16)
        rvec = lax.dot_general(m2f * a, h, (((0,), (0,)), ((), ())),
                               preferred_element_type=jnp.float32)
        q_star = jnp.concatenate([qh, rvec], axis=1)
    og2 = jnp.concatenate([q_star, ga_ref[...]], axis=1)
    og2 = jnp.maximum(
        jnp.dot(og2, w1_ref[...], preferred_element_type=jnp.float32)
        + b1_ref[...], 0.0)
    o_ref[...] = (jnp.dot(og2, w2_ref[...], preferred_element_type=jnp.float32)
                  + b2_ref[...])


def _sc_mesh():
    return plsc.VectorSubcoreMesh(core_axis_name="c", subcore_axis_name="s")


@functools.lru_cache(maxsize=None)
def _make_gather(ep):
    nchunk = ep // (CH * NW)

    @functools.partial(
        pl.kernel,
        out_type=jax.ShapeDtypeStruct((ep, DP), jnp.float32),
        mesh=_sc_mesh(),
        scratch_types=[pltpu.VMEM((CH,), jnp.int32),
                       pltpu.VMEM((CH, DP), jnp.float32),
                       pltpu.SemaphoreType.DMA],
    )
    def gather_k(table_hbm, src_hbm, out_hbm, idx_v, rows_v, sem):
        wid = lax.axis_index("s") * NC + lax.axis_index("c")

        def body(r, carry):
            off = (r * NW + wid) * CH
            pltpu.sync_copy(src_hbm.at[pl.ds(off, CH)], idx_v)
            pltpu.async_copy(table_hbm.at[idx_v], rows_v, sem).wait()
            pltpu.sync_copy(rows_v, out_hbm.at[pl.ds(off, CH)])
            return carry

        lax.fori_loop(0, nchunk, body, 0)

    return gather_k


@functools.lru_cache(maxsize=None)
def _make_scatter(ep, nr):
    nchunk = ep // (CH * NW)
    rps = nr // NS

    @functools.partial(
        pl.kernel,
        out_type=jax.ShapeDtypeStruct((NC * nr, DP), jnp.float32),
        mesh=_sc_mesh(),
        scratch_types=[pltpu.VMEM((CH,), jnp.int32),
                       pltpu.VMEM((CH, DP), jnp.float32),
                       pltpu.VMEM_SHARED((nr, DP), jnp.float32),
                       pltpu.SemaphoreType.DMA],
    )
    def scatter_k(msg_hbm, dst_hbm, zeros_hbm, out_hbm, idx_v, rows_v, shared, sem):
        c = lax.axis_index("c")
        s = lax.axis_index("s")
        wid = s * NC + c
        pltpu.sync_copy(zeros_hbm.at[pl.ds(s * rps, rps)],
                        shared.at[pl.ds(s * rps, rps)])
        plsc.subcore_barrier()

        def body(r, carry):
            off = (r * NW + wid) * CH
            pltpu.sync_copy(dst_hbm.at[pl.ds(off, CH)], idx_v)
            pltpu.sync_copy(msg_hbm.at[pl.ds(off, CH)], rows_v)
            pltpu.sync_copy(rows_v, shared.at[idx_v], add=True)
            return carry

        lax.fori_loop(0, nchunk, body, 0)
        plsc.subcore_barrier()
        pltpu.sync_copy(shared.at[pl.ds(s * rps, rps)],
                        out_hbm.at[pl.ds(c * nr + s * rps, rps)])

    return scatter_k


@functools.lru_cache(maxsize=None)
def _make_degree(ep, nr):
    nchunk = ep // (CH * NW)
    rps = nr // NS

    @functools.partial(
        pl.kernel,
        out_type=jax.ShapeDtypeStruct((NC * nr, DP), jnp.float32),
        mesh=_sc_mesh(),
        scratch_types=[pltpu.VMEM((CH,), jnp.int32),
                       pltpu.VMEM((CH, DP), jnp.float32),
                       pltpu.VMEM_SHARED((nr, DP), jnp.float32),
                       pltpu.SemaphoreType.DMA],
    )
    def degree_k(dst_hbm, zeros_hbm, ones_hbm, out_hbm, idx_v, rows_v, shared, sem):
        c = lax.axis_index("c")
        s = lax.axis_index("s")
        wid = s * NC + c
        pltpu.sync_copy(zeros_hbm.at[pl.ds(s * rps, rps)],
                        shared.at[pl.ds(s * rps, rps)])
        pltpu.sync_copy(ones_hbm, rows_v)
        plsc.subcore_barrier()

        def body(r, carry):
            off = (r * NW + wid) * CH
            pltpu.sync_copy(dst_hbm.at[pl.ds(off, CH)], idx_v)
            pltpu.sync_copy(rows_v, shared.at[idx_v], add=True)
            return carry

        lax.fori_loop(0, nchunk, body, 0)
        plsc.subcore_barrier()
        pltpu.sync_copy(shared.at[pl.ds(s * rps, rps)],
                        out_hbm.at[pl.ds(c * nr + s * rps, rps)])

    return degree_k


def kernel(x, edge_index, edge_attr, batch, graph_attr, W0, b0, We1, be1,
           We2, be2, Wroot, bconv, gru_Wih, gru_Whh, gru_bih, gru_bhh,
           lstm_Wih, lstm_Whh, lstm_bih, lstm_bhh, W1, b1, W2, b2):
    n, df = x.shape
    e = edge_index.shape[1]
    de = edge_attr.shape[1]
    ng, dg = graph_attr.shape

    ep = -(-e // (CH * NW)) * (CH * NW)
    nr = -(-(n + 1) // 128) * 128

    src_p = jnp.concatenate([edge_index[0], jnp.zeros((ep - e,), jnp.int32)])
    dst_p = jnp.concatenate([edge_index[1], jnp.full((ep - e,), n, jnp.int32)])
    ea_p = jnp.concatenate([edge_attr, jnp.zeros((ep - e, de), jnp.float32)])
    zeros_nr = jnp.zeros((nr, DP), jnp.float32)
    ones_ch = jnp.ones((CH, DP), jnp.float32)

    w0_p = jnp.concatenate([W0, jnp.zeros((df, DP - DC), jnp.float32)], axis=1)
    b0_p = jnp.concatenate([b0, jnp.zeros((DP - DC,), jnp.float32)]).reshape(1, DP)

    bl = 1000
    nb = n // bl
    bke = 1024
    neb = ep // bke

    h = pl.pallas_call(
        _lin0_body,
        grid=(nb,),
        in_specs=[pl.BlockSpec((bl, df), lambda i: (i, 0)),
                  pl.BlockSpec((df, DP), lambda i: (0, 0)),
                  pl.BlockSpec((1, DP), lambda i: (0, 0))],
        out_specs=pl.BlockSpec((bl, DP), lambda i: (i, 0)),
        out_shape=jax.ShapeDtypeStruct((n, DP), jnp.float32),
    )(x, w0_p, b0_p)

    perm = (jnp.arange(DC * DC) % DC) * DC + jnp.arange(DC * DC) // DC
    we2g = We2[:, perm]
    be2g = be2[perm]
    sel = jnp.kron(jnp.eye(DC, dtype=jnp.float32), jnp.ones((DC, 1), jnp.float32))
    ew = pl.pallas_call(
        _ew_body,
        grid=(neb,),
        in_specs=[pl.BlockSpec((bke, de), lambda i: (i, 0)),
                  pl.BlockSpec((de, We1.shape[1]), lambda i: (0, 0)),
                  pl.BlockSpec((1, We1.shape[1]), lambda i: (0, 0)),
                  pl.BlockSpec((We2.shape[0], DC * DC), lambda i: (0, 0)),
                  pl.BlockSpec((1, DC * DC), lambda i: (0, 0))],
        out_specs=pl.BlockSpec((bke, DC * DC), lambda i: (i, 0)),
        out_shape=jax.ShapeDtypeStruct((ep, DC * DC), jnp.bfloat16),
    )(ea_p, We1, be1.reshape(1, -1), we2g, be2g.reshape(1, -1))

    deg = _make_degree(ep, nr)(dst_p, zeros_nr, ones_ch).reshape(NC, nr, DP)

    gather_k = _make_gather(ep)
    scatter_k = _make_scatter(ep, nr)

    wih_t = gru_Wih.T
    whh_t = gru_Whh.T
    gbih = gru_bih.reshape(1, -1)
    gbhh = gru_bhh.reshape(1, -1)

    for _ in range(3):
        s_rows = gather_k(h, src_p)
        msg = pl.pallas_call(
            _msg_body,
            grid=(neb,),
            in_specs=[pl.BlockSpec((bke, DP), lambda i: (i, 0)),
                      pl.BlockSpec((bke, DC * DC), lambda i: (i, 0)),
                      pl.BlockSpec((DC * DC, DC), lambda i: (0, 0))],
            out_specs=pl.BlockSpec((bke, DP), lambda i: (i, 0)),
            out_shape=jax.ShapeDtypeStruct((ep, DP), jnp.float32),
        )(s_rows, ew, sel)
        agg = scatter_k(msg, dst_p, zeros_nr).reshape(NC, nr, DP)
        h = pl.pallas_call(
            _gru_body,
            grid=(nb,),
            in_specs=[pl.BlockSpec((bl, DP), lambda i: (i, 0)),
                      pl.BlockSpec((NC, bl, DP), lambda i: (0, i, 0)),
                      pl.BlockSpec((NC, bl, DP), lambda i: (0, i, 0)),
                      pl.BlockSpec((DC, DC), lambda i: (0, 0)),
                      pl.BlockSpec((1, DC), lambda i: (0, 0)),
                      pl.BlockSpec((DC, 3 * DC), lambda i: (0, 0)),
                      pl.BlockSpec((DC, 3 * DC), lambda i: (0, 0)),
                      pl.BlockSpec((1, 3 * DC), lambda i: (0, 0)),
                      pl.BlockSpec((1, 3 * DC), lambda i: (0, 0))],
            out_specs=pl.BlockSpec((bl, DP), lambda i: (i, 0)),
            out_shape=jax.ShapeDtypeStruct((n, DP), jnp.float32),
        )(h, agg, deg, Wroot, bconv.reshape(1, DC), wih_t, whh_t, gbih, gbhh)

    out = pl.pallas_call(
        _s2s_body,
        out_shape=jax.ShapeDtypeStruct((ng, 1), jnp.float32),
    )(h, batch.reshape(n, 1), graph_attr, lstm_Wih.T, lstm_Whh.T,
      lstm_bih.reshape(1, -1), lstm_bhh.reshape(1, -1),
      W1, b1.reshape(1, -1), W2, b2.reshape(1, 1))
    return out


# speedup vs baseline: 1.1422x; 1.0252x over previous
import functools

import jax
import jax.numpy as jnp
from jax import lax
from jax.experimental import pallas as pl
from jax.experimental.pallas import tpu as pltpu
from jax.experimental.pallas import tpu_sc as plsc

NC = 2
NS = 16
NW = NC * NS
CH = 128
DC = 32
DP = 128


def _lin0_body(x_ref, w_ref, b_ref, o_ref):
    o_ref[...] = jnp.maximum(
        jnp.dot(x_ref[...], w_ref[...], preferred_element_type=jnp.float32)
        + b_ref[...], 0.0)


def _ew_body(ea_ref, we1_ref, be1_ref, we2_ref, be2_ref, o_ref):
    h1 = jnp.maximum(
        jnp.dot(ea_ref[...], we1_ref[...], preferred_element_type=jnp.float32)
        + be1_ref[...], 0.0)
    o_ref[...] = (jnp.dot(h1, we2_ref[...], preferred_element_type=jnp.float32)
                  + be2_ref[...]).astype(jnp.bfloat16)


def _msg_body(s_ref, ew_ref, sel_ref, o_ref):
    s = s_ref[:, 0:DC]
    srep = jnp.concatenate([s] * DC, axis=1)
    t = ew_ref[...].astype(jnp.float32) * srep
    msg = jnp.dot(t, sel_ref[...], preferred_element_type=jnp.float32)
    o_ref[...] = jnp.concatenate(
        [msg, jnp.zeros((msg.shape[0], DP - DC), jnp.float32)], axis=1)


def _gru_body(h_ref, agg_ref, deg_ref, wroot_ref, bconv_ref,
              wih_ref, whh_ref, bih_ref, bhh_ref, o_ref):
    h = h_ref[:, 0:DC]
    agg = agg_ref[0, :, 0:DC] + agg_ref[1, :, 0:DC]
    deg = jnp.maximum(deg_ref[0, :, 0:DC] + deg_ref[1, :, 0:DC], 1.0)
    m = jnp.maximum(
        jnp.dot(h, wroot_ref[...], preferred_element_type=jnp.float32)
        + agg / deg + bconv_ref[...], 0.0)
    gi = jnp.dot(m, wih_ref[...], preferred_element_type=jnp.float32) + bih_ref[...]
    gh = jnp.dot(h, whh_ref[...], preferred_element_type=jnp.float32) + bhh_ref[...]
    r = jax.nn.sigmoid(gi[:, 0:DC] + gh[:, 0:DC])
    z = jax.nn.sigmoid(gi[:, DC:2 * DC] + gh[:, DC:2 * DC])
    ng = jnp.tanh(gi[:, 2 * DC:3 * DC] + r * gh[:, 2 * DC:3 * DC])
    hn = (1.0 - z) * ng + z * h
    o_ref[...] = jnp.concatenate(
        [hn, jnp.zeros((hn.shape[0], DP - DC), jnp.float32)], axis=1)


def _s2s_body(h_ref, bcol_ref, ga_ref, wih_ref, whh_ref, bih_ref, bhh_ref,
              w1_ref, b1_ref, w2_ref, b2_ref, o_ref):
    h = h_ref[:, 0:DC]
    n = h.shape[0]
    ng = ga_ref.shape[0]
    m2 = bcol_ref[...] == lax.broadcasted_iota(jnp.int32, (n, ng), 1)
    m2f = m2.astype(jnp.float32)
    qh = jnp.zeros((ng, DC), jnp.float32)
    qc = jnp.zeros((ng, DC), jnp.float32)
    q_star = jnp.zeros((ng, 2 * DC), jnp.float32)
    for _ in range(3):
        gates = (jnp.dot(q_star, wih_ref[...], preferred_element_type=jnp.float32)
                 + bih_ref[...]
                 + jnp.dot(qh, whh_ref[...], preferred_element_type=jnp.float32)
                 + bhh_ref[...])
        ig = jax.nn.sigmoid(gates[:, 0:DC])
        fg = jax.nn.sigmoid(gates[:, DC:2 * DC])
        gg = jnp.tanh(gates[:, 2 * DC:3 * DC])
        og = jax.nn.sigmoid(gates[:, 3 * DC:4 * DC])
        qc = fg * qc + ig * gg
        qh = og * jnp.tanh(qc)
        e2 = lax.dot_general(h, qh, (((1,), (1,)), ((), ())),
                             preferred_element_type=jnp.float32)
        e = jnp.sum(jnp.where(m2, e2, 0.0), axis=1, keepdims=True)
        emax = jnp.max(jnp.where(m2, e, -1e30), axis=0, keepdims=True)
        ee = jnp.exp(e - jnp.sum(m2f * emax, axis=1, keepdims=True))
        denom = jnp.sum(m2f * ee, axis=0, keepdims=True)
        a = ee / (jnp.sum(m2f * denom, axis=1, keepdims=True) + 1e-16)
        rvec = lax.dot_general(m2f * a, h, (((0,), (0,)), ((), ())),
                               preferred_element_type=jnp.float32)
        q_star = jnp.concatenate([qh, rvec], axis=1)
    og2 = jnp.concatenate([q_star, ga_ref[...]], axis=1)
    og2 = jnp.maximum(
        jnp.dot(og2, w1_ref[...], preferred_element_type=jnp.float32)
        + b1_ref[...], 0.0)
    o_ref[...] = (jnp.dot(og2, w2_ref[...], preferred_element_type=jnp.float32)
                  + b2_ref[...])


def _sc_mesh():
    return plsc.VectorSubcoreMesh(core_axis_name="c", subcore_axis_name="s")


@functools.lru_cache(maxsize=None)
def _make_gather(ep):
    nchunk = ep // (CH * NW)

    @functools.partial(
        pl.kernel,
        out_type=jax.ShapeDtypeStruct((ep, DP), jnp.float32),
        mesh=_sc_mesh(),
        scratch_types=[pltpu.VMEM((CH,), jnp.int32),
                       pltpu.VMEM((CH, DP), jnp.float32),
                       pltpu.SemaphoreType.DMA],
    )
    def gather_k(table_hbm, src_hbm, out_hbm, idx_v, rows_v, sem):
        wid = lax.axis_index("s") * NC + lax.axis_index("c")

        def body(r, carry):
            off = (r * NW + wid) * CH
            pltpu.sync_copy(src_hbm.at[pl.ds(off, CH)], idx_v)
            pltpu.async_copy(table_hbm.at[idx_v], rows_v, sem).wait()
            pltpu.sync_copy(rows_v, out_hbm.at[pl.ds(off, CH)])
            return carry

        lax.fori_loop(0, nchunk, body, 0)

    return gather_k


@functools.lru_cache(maxsize=None)
def _make_scatter(ep, nr):
    nchunk = ep // (CH * NW)
    rps = nr // NS

    @functools.partial(
        pl.kernel,
        out_type=jax.ShapeDtypeStruct((NC * nr, DP), jnp.float32),
        mesh=_sc_mesh(),
        scratch_types=[pltpu.VMEM((CH,), jnp.int32),
                       pltpu.VMEM((CH, DP), jnp.float32),
                       pltpu.VMEM_SHARED((nr, DP), jnp.float32),
                       pltpu.SemaphoreType.DMA,
                       pltpu.SemaphoreType.DMA],
    )
    def scatter_k(msg_hbm, dst_hbm, zeros_hbm, out_hbm, idx_v, rows_v, shared,
                  sem, sem2):
        c = lax.axis_index("c")
        s = lax.axis_index("s")
        wid = s * NC + c
        pltpu.sync_copy(zeros_hbm.at[pl.ds(s * rps, rps)],
                        shared.at[pl.ds(s * rps, rps)])
        plsc.subcore_barrier()

        def body(r, carry):
            off = (r * NW + wid) * CH
            a = pltpu.async_copy(dst_hbm.at[pl.ds(off, CH)], idx_v, sem)
            b = pltpu.async_copy(msg_hbm.at[pl.ds(off, CH)], rows_v, sem2)
            a.wait()
            b.wait()
            pltpu.sync_copy(rows_v, shared.at[idx_v], add=True)
            return carry

        lax.fori_loop(0, nchunk, body, 0)
        plsc.subcore_barrier()
        pltpu.sync_copy(shared.at[pl.ds(s * rps, rps)],
                        out_hbm.at[pl.ds(c * nr + s * rps, rps)])

    return scatter_k


@functools.lru_cache(maxsize=None)
def _make_degree(ep, nr):
    nchunk = ep // (CH * NW)
    rps = nr // NS

    @functools.partial(
        pl.kernel,
        out_type=jax.ShapeDtypeStruct((NC * nr, DP), jnp.float32),
        mesh=_sc_mesh(),
        scratch_types=[pltpu.VMEM((CH,), jnp.int32),
                       pltpu.VMEM((CH, DP), jnp.float32),
                       pltpu.VMEM_SHARED((nr, DP), jnp.float32),
                       pltpu.SemaphoreType.DMA],
    )
    def degree_k(dst_hbm, zeros_hbm, ones_hbm, out_hbm, idx_v, rows_v, shared, sem):
        c = lax.axis_index("c")
        s = lax.axis_index("s")
        wid = s * NC + c
        pltpu.sync_copy(zeros_hbm.at[pl.ds(s * rps, rps)],
                        shared.at[pl.ds(s * rps, rps)])
        pltpu.sync_copy(ones_hbm, rows_v)
        plsc.subcore_barrier()

        def body(r, carry):
            off = (r * NW + wid) * CH
            pltpu.sync_copy(dst_hbm.at[pl.ds(off, CH)], idx_v)
            pltpu.sync_copy(rows_v, shared.at[idx_v], add=True)
            return carry

        lax.fori_loop(0, nchunk, body, 0)
        plsc.subcore_barrier()
        pltpu.sync_copy(shared.at[pl.ds(s * rps, rps)],
                        out_hbm.at[pl.ds(c * nr + s * rps, rps)])

    return degree_k


def kernel(x, edge_index, edge_attr, batch, graph_attr, W0, b0, We1, be1,
           We2, be2, Wroot, bconv, gru_Wih, gru_Whh, gru_bih, gru_bhh,
           lstm_Wih, lstm_Whh, lstm_bih, lstm_bhh, W1, b1, W2, b2):
    n, df = x.shape
    e = edge_index.shape[1]
    de = edge_attr.shape[1]
    ng, dg = graph_attr.shape

    ep = -(-e // (CH * NW)) * (CH * NW)
    nr = -(-(n + 1) // 128) * 128

    src_p = jnp.concatenate([edge_index[0], jnp.zeros((ep - e,), jnp.int32)])
    dst_p = jnp.concatenate([edge_index[1], jnp.full((ep - e,), n, jnp.int32)])
    ea_p = jnp.concatenate([edge_attr, jnp.zeros((ep - e, de), jnp.float32)])
    zeros_nr = jnp.zeros((nr, DP), jnp.float32)
    ones_ch = jnp.ones((CH, DP), jnp.float32)

    w0_p = jnp.concatenate([W0, jnp.zeros((df, DP - DC), jnp.float32)], axis=1)
    b0_p = jnp.concatenate([b0, jnp.zeros((DP - DC,), jnp.float32)]).reshape(1, DP)

    bl = 1000
    nb = n // bl
    bke = 1024
    neb = ep // bke

    h = pl.pallas_call(
        _lin0_body,
        grid=(nb,),
        in_specs=[pl.BlockSpec((bl, df), lambda i: (i, 0)),
                  pl.BlockSpec((df, DP), lambda i: (0, 0)),
                  pl.BlockSpec((1, DP), lambda i: (0, 0))],
        out_specs=pl.BlockSpec((bl, DP), lambda i: (i, 0)),
        out_shape=jax.ShapeDtypeStruct((n, DP), jnp.float32),
    )(x, w0_p, b0_p)

    perm = (jnp.arange(DC * DC) % DC) * DC + jnp.arange(DC * DC) // DC
    we2g = We2[:, perm]
    be2g = be2[perm]
    sel = jnp.kron(jnp.eye(DC, dtype=jnp.float32), jnp.ones((DC, 1), jnp.float32))
    ew = pl.pallas_call(
        _ew_body,
        grid=(neb,),
        in_specs=[pl.BlockSpec((bke, de), lambda i: (i, 0)),
                  pl.BlockSpec((de, We1.shape[1]), lambda i: (0, 0)),
                  pl.BlockSpec((1, We1.shape[1]), lambda i: (0, 0)),
                  pl.BlockSpec((We2.shape[0], DC * DC), lambda i: (0, 0)),
                  pl.BlockSpec((1, DC * DC), lambda i: (0, 0))],
        out_specs=pl.BlockSpec((bke, DC * DC), lambda i: (i, 0)),
        out_shape=jax.ShapeDtypeStruct((ep, DC * DC), jnp.bfloat16),
    )(ea_p, We1, be1.reshape(1, -1), we2g, be2g.reshape(1, -1))

    deg = _make_degree(ep, nr)(dst_p, zeros_nr, ones_ch).reshape(NC, nr, DP)

    gather_k = _make_gather(ep)
    scatter_k = _make_scatter(ep, nr)

    wih_t = gru_Wih.T
    whh_t = gru_Whh.T
    gbih = gru_bih.reshape(1, -1)
    gbhh = gru_bhh.reshape(1, -1)

    for _ in range(3):
        s_rows = gather_k(h, src_p)
        msg = pl.pallas_call(
            _msg_body,
            grid=(neb,),
            in_specs=[pl.BlockSpec((bke, DP), lambda i: (i, 0)),
                      pl.BlockSpec((bke, DC * DC), lambda i: (i, 0)),
                      pl.BlockSpec((DC * DC, DC), lambda i: (0, 0))],
            out_specs=pl.BlockSpec((bke, DP), lambda i: (i, 0)),
            out_shape=jax.ShapeDtypeStruct((ep, DP), jnp.float32),
        )(s_rows, ew, sel)
        agg = scatter_k(msg, dst_p, zeros_nr).reshape(NC, nr, DP)
        h = pl.pallas_call(
            _gru_body,
            grid=(nb,),
            in_specs=[pl.BlockSpec((bl, DP), lambda i: (i, 0)),
                      pl.BlockSpec((NC, bl, DP), lambda i: (0, i, 0)),
                      pl.BlockSpec((NC, bl, DP), lambda i: (0, i, 0)),
                      pl.BlockSpec((DC, DC), lambda i: (0, 0)),
                      pl.BlockSpec((1, DC), lambda i: (0, 0)),
                      pl.BlockSpec((DC, 3 * DC), lambda i: (0, 0)),
                      pl.BlockSpec((DC, 3 * DC), lambda i: (0, 0)),
                      pl.BlockSpec((1, 3 * DC), lambda i: (0, 0)),
                      pl.BlockSpec((1, 3 * DC), lambda i: (0, 0))],
            out_specs=pl.BlockSpec((bl, DP), lambda i: (i, 0)),
            out_shape=jax.ShapeDtypeStruct((n, DP), jnp.float32),
        )(h, agg, deg, Wroot, bconv.reshape(1, DC), wih_t, whh_t, gbih, gbhh)

    out = pl.pallas_call(
        _s2s_body,
        out_shape=jax.ShapeDtypeStruct((ng, 1), jnp.float32),
    )(h, batch.reshape(n, 1), graph_attr, lstm_Wih.T, lstm_Whh.T,
      lstm_bih.reshape(1, -1), lstm_bhh.reshape(1, -1),
      W1, b1.reshape(1, -1), W2, b2.reshape(1, 1))
    return out
